# Initial kernel scaffold; baseline (speedup 1.0000x reference)
import jax
import jax.numpy as jnp
from jax.experimental import pallas as pl


def kernel(Z, weight):
    raise NotImplementedError("write your pallas kernel here")


# speedup vs baseline: 1.4741x; 1.4741x over previous
import jax
import jax.numpy as jnp
from jax import lax
from jax.experimental import pallas as pl
from jax.experimental.pallas import tpu as pltpu
from jax.experimental.pallas import tpu_sc as plsc

_N_ATOMS = 100000
_EMB = 128
_NC = 2
_NS = 16
_NW = _NC * _NS
_CHUNK = 200
_NCHUNKS = _N_ATOMS // _CHUNK
_STEPS = -(-_NCHUNKS // _NW)


def _emb_body(idx_hbm, table_hbm, out_hbm, idx_v, rows_v, sem):
    wid = lax.axis_index("s") * _NC + lax.axis_index("c")

    def step(c, carry):
        cid = c * _NW + wid

        @pl.when(cid < _NCHUNKS)
        def _():
            base = cid * _CHUNK
            pltpu.sync_copy(idx_hbm.at[pl.ds(base, _CHUNK)], idx_v)
            pltpu.async_copy(table_hbm.at[idx_v], rows_v, sem).wait()
            pltpu.sync_copy(rows_v, out_hbm.at[pl.ds(base, _CHUNK)])

        return carry

    lax.fori_loop(0, _STEPS, step, 0)


_emb_call = pl.kernel(
    _emb_body,
    out_type=jax.ShapeDtypeStruct((_N_ATOMS, _EMB), jnp.float32),
    mesh=plsc.VectorSubcoreMesh(core_axis_name="c", subcore_axis_name="s"),
    scratch_types=[
        pltpu.VMEM((_CHUNK,), jnp.int32),
        pltpu.VMEM((_CHUNK, _EMB), jnp.float32),
        pltpu.SemaphoreType.DMA,
    ],
)


def kernel(Z, weight):
    idx = (Z - 1).astype(jnp.int32)
    return _emb_call(idx, weight)


# speedup vs baseline: 1.4746x; 1.0004x over previous
import jax
import jax.numpy as jnp
from jax import lax
from jax.experimental import pallas as pl
from jax.experimental.pallas import tpu as pltpu
from jax.experimental.pallas import tpu_sc as plsc

_N_ATOMS = 100000
_EMB = 128
_NC = 2
_NS = 16
_NW = _NC * _NS
_CHUNK = 400
_NCHUNKS = _N_ATOMS // _CHUNK
_STEPS = -(-_NCHUNKS // _NW)
_LAST_FULL = _NCHUNKS - (_STEPS - 1) * _NW


def _emb_body(idx_hbm, table_hbm, out_hbm, idx_v0, idx_v1, rows_v0, rows_v1,
              isem, gsem, osem):
    wid = lax.axis_index("s") * _NC + lax.axis_index("c")
    idx_bufs = (idx_v0, idx_v1)
    rows_bufs = (rows_v0, rows_v1)

    def cid(c):
        return jnp.minimum(c * _NW + wid, _NCHUNKS - 1)

    def idx_copy(c):
        return pltpu.make_async_copy(idx_hbm.at[cid(c)], idx_bufs[c % 2], isem)

    def gather(c):
        return pltpu.make_async_copy(
            table_hbm.at[idx_bufs[c % 2]], rows_bufs[c % 2], gsem)

    def writeout(c):
        return pltpu.make_async_copy(
            rows_bufs[c % 2], out_hbm.at[pl.ds(cid(c) * _CHUNK, _CHUNK)], osem)

    idx_copy(0).start()
    for c in range(_STEPS - 1):
        idx_copy(c).wait()
        if c >= 2:
            writeout(c - 2).wait()
        gather(c).start()
        if c >= 1:
            gather(c - 1).wait()
            writeout(c - 1).start()
        idx_copy(c + 1).start()

    last = _STEPS - 1
    writeout(last - 2).wait()
    idx_copy(last).wait()
    gather(last - 1).wait()
    writeout(last - 1).start()

    @pl.when(wid < _LAST_FULL)
    def _():
        gather(last).start()
        gather(last).wait()
        writeout(last).start()
        writeout(last).wait()

    writeout(last - 1).wait()


_emb_call = pl.kernel(
    _emb_body,
    out_type=jax.ShapeDtypeStruct((_N_ATOMS, _EMB), jnp.float32),
    mesh=plsc.VectorSubcoreMesh(core_axis_name="c", subcore_axis_name="s"),
    scratch_types=[
        pltpu.VMEM((_CHUNK,), jnp.int32),
        pltpu.VMEM((_CHUNK,), jnp.int32),
        pltpu.VMEM((_CHUNK, _EMB), jnp.float32),
        pltpu.VMEM((_CHUNK, _EMB), jnp.float32),
        pltpu.SemaphoreType.DMA,
        pltpu.SemaphoreType.DMA,
        pltpu.SemaphoreType.DMA,
    ],
)


def kernel(Z, weight):
    idx = (Z - 1).astype(jnp.int32).reshape(_NCHUNKS, _CHUNK)
    return _emb_call(idx, weight)


# speedup vs baseline: 5.6411x; 3.8254x over previous
import jax
import jax.numpy as jnp
from jax import lax
from jax.experimental import pallas as pl
from jax.experimental.pallas import tpu as pltpu
from jax.experimental.pallas import tpu_sc as plsc

_N_ATOMS = 100000
_EMB = 128
_NC = 2
_NS = 16
_NW = _NC * _NS
_CHUNK = 400
_NCHUNKS = _N_ATOMS // _CHUNK
_STEPS = -(-_NCHUNKS // _NW)
_LAST_FULL = _NCHUNKS - (_STEPS - 1) * _NW


def _emb_body(idx_hbm, table_hbm, out_hbm, table_v, idx_v0, idx_v1,
              rows_v0, rows_v1, isem, gsem, osem):
    wid = lax.axis_index("s") * _NC + lax.axis_index("c")
    idx_bufs = (idx_v0, idx_v1)
    rows_bufs = (rows_v0, rows_v1)
    @pl.when(lax.axis_index("s") == 0)
    def _():
        pltpu.sync_copy(table_hbm, table_v)

    plsc.subcore_barrier()

    def cid(c):
        return jnp.minimum(c * _NW + wid, _NCHUNKS - 1)

    def idx_copy(c):
        return pltpu.make_async_copy(idx_hbm.at[cid(c)], idx_bufs[c % 2], isem)

    def gather(c):
        return pltpu.make_async_copy(
            table_v.at[idx_bufs[c % 2]], rows_bufs[c % 2], gsem)

    def writeout(c):
        return pltpu.make_async_copy(
            rows_bufs[c % 2], out_hbm.at[pl.ds(cid(c) * _CHUNK, _CHUNK)], osem)

    idx_copy(0).start()
    for c in range(_STEPS - 1):
        idx_copy(c).wait()
        if c >= 2:
            writeout(c - 2).wait()
        gather(c).start()
        if c >= 1:
            gather(c - 1).wait()
            writeout(c - 1).start()
        idx_copy(c + 1).start()

    last = _STEPS - 1
    writeout(last - 2).wait()
    idx_copy(last).wait()
    gather(last - 1).wait()
    writeout(last - 1).start()

    @pl.when(wid < _LAST_FULL)
    def _():
        gather(last).start()
        gather(last).wait()
        writeout(last).start()
        writeout(last).wait()

    writeout(last - 1).wait()


_emb_call = pl.kernel(
    _emb_body,
    out_type=jax.ShapeDtypeStruct((_N_ATOMS, _EMB), jnp.float32),
    mesh=plsc.VectorSubcoreMesh(core_axis_name="c", subcore_axis_name="s"),
    scratch_types=[
        pltpu.VMEM_SHARED((100, _EMB), jnp.float32),
        pltpu.VMEM((_CHUNK,), jnp.int32),
        pltpu.VMEM((_CHUNK,), jnp.int32),
        pltpu.VMEM((_CHUNK, _EMB), jnp.float32),
        pltpu.VMEM((_CHUNK, _EMB), jnp.float32),
        pltpu.SemaphoreType.DMA,
        pltpu.SemaphoreType.DMA,
        pltpu.SemaphoreType.DMA,
    ],
)


def kernel(Z, weight):
    idx = (Z - 1).astype(jnp.int32).reshape(_NCHUNKS, _CHUNK)
    return _emb_call(idx, weight)


# speedup vs baseline: 5.6588x; 1.0031x over previous
import jax
import jax.numpy as jnp
from jax import lax
from jax.experimental import pallas as pl
from jax.experimental.pallas import tpu as pltpu
from jax.experimental.pallas import tpu_sc as plsc

_N_ATOMS = 100000
_EMB = 128
_NC = 2
_NS = 16
_NW = _NC * _NS
_CHUNK = 400
_NCHUNKS = _N_ATOMS // _CHUNK
_STEPS = -(-_NCHUNKS // _NW)
_LAST_FULL = _NCHUNKS - (_STEPS - 1) * _NW


def _emb_body(idx_hbm, table_hbm, out_hbm, table_v, idx_v0, idx_v1,
              rows_v0, rows_v1, isem, gsem, osem):
    wid = lax.axis_index("s") * _NC + lax.axis_index("c")
    idx_bufs = (idx_v0, idx_v1)
    rows_bufs = (rows_v0, rows_v1)
    @pl.when(lax.axis_index("s") == 0)
    def _():
        pltpu.sync_copy(table_hbm, table_v.at[pl.ds(1, 100)])

    plsc.subcore_barrier()

    def cid(c):
        return jnp.minimum(c * _NW + wid, _NCHUNKS - 1)

    def idx_copy(c):
        return pltpu.make_async_copy(
            idx_hbm.at[pl.ds(cid(c) * _CHUNK, _CHUNK)], idx_bufs[c % 2], isem)

    def gather(c):
        return pltpu.make_async_copy(
            table_v.at[idx_bufs[c % 2]], rows_bufs[c % 2], gsem)

    def writeout(c):
        return pltpu.make_async_copy(
            rows_bufs[c % 2], out_hbm.at[pl.ds(cid(c) * _CHUNK, _CHUNK)], osem)

    idx_copy(0).start()
    for c in range(_STEPS - 1):
        idx_copy(c).wait()
        if c >= 2:
            writeout(c - 2).wait()
        gather(c).start()
        if c >= 1:
            gather(c - 1).wait()
            writeout(c - 1).start()
        idx_copy(c + 1).start()

    last = _STEPS - 1
    writeout(last - 2).wait()
    idx_copy(last).wait()
    gather(last - 1).wait()
    writeout(last - 1).start()

    @pl.when(wid < _LAST_FULL)
    def _():
        gather(last).start()
        gather(last).wait()
        writeout(last).start()
        writeout(last).wait()

    writeout(last - 1).wait()


_emb_call = pl.kernel(
    _emb_body,
    out_type=jax.ShapeDtypeStruct((_N_ATOMS, _EMB), jnp.float32),
    mesh=plsc.VectorSubcoreMesh(core_axis_name="c", subcore_axis_name="s"),
    scratch_types=[
        pltpu.VMEM_SHARED((104, _EMB), jnp.float32),
        pltpu.VMEM((_CHUNK,), jnp.int32),
        pltpu.VMEM((_CHUNK,), jnp.int32),
        pltpu.VMEM((_CHUNK, _EMB), jnp.float32),
        pltpu.VMEM((_CHUNK, _EMB), jnp.float32),
        pltpu.SemaphoreType.DMA,
        pltpu.SemaphoreType.DMA,
        pltpu.SemaphoreType.DMA,
    ],
)


def kernel(Z, weight):
    return _emb_call(Z, weight)
